# baseline (speedup 1.0000x reference)
import functools

import jax
import jax.numpy as jnp
from jax import lax
from jax.experimental import pallas as pl
from jax.experimental.pallas import tpu as pltpu
from jax.experimental.pallas import tpu_sc as plsc

N_NODES = 10000
N_PAD = 10240
E_TOTAL = 320000
D = 128
NW = 32
EPW = E_TOTAL // NW
CH = 40
NCH = EPW // CH
NPT = N_PAD // 16

_SRC_TAB = (1, 0, 0, 0, 1)


NBUF = 5
NGRP = NCH // NBUF


def _sc_body(op_feats, device_feats,
             src0, dst0, ef0, src1, dst1, ef1, src2, dst2, ef2,
             src3, dst3, ef3, src4, dst4, ef4,
             acc_s_out, acc_e_out, *scr):
    acc_s, acc_e = scr[0], scr[1]
    srcv = scr[2:2 + NBUF]
    dstv = scr[2 + NBUF:2 + 2 * NBUF]
    rows = scr[2 + 2 * NBUF:2 + 3 * NBUF]
    aug = scr[2 + 3 * NBUF:2 + 4 * NBUF]
    sems = scr[2 + 4 * NBUF:2 + 5 * NBUF]

    c = lax.axis_index("c")
    s = lax.axis_index("s")
    wid = c * 16 + s
    ebase = wid * EPW
    nbase = s * NPT

    zero16 = jnp.zeros((16,), jnp.float32)

    edge_lists = ((src0, dst0, ef0), (src1, dst1, ef1), (src2, dst2, ef2),
                  (src3, dst3, ef3), (src4, dst4, ef4))

    for et in range(5):
        src_h, dst_h, ef_h = edge_lists[et]
        feats_h = device_feats if _SRC_TAB[et] else op_feats

        def _zrow(i, carry):
            for j in range(8):
                rows[0][i, pl.ds(j * 16, 16)] = zero16
            aug[0][i, :] = zero16
            return carry
        lax.fori_loop(0, CH, _zrow, 0)
        for z in range(NPT // CH):
            pltpu.sync_copy(rows[0], acc_s.at[pl.ds(nbase + z * CH, CH), :])
            pltpu.sync_copy(aug[0], acc_e.at[pl.ds(nbase + z * CH, CH), :])
        plsc.subcore_barrier()

        for b in range(NBUF):
            eb = ebase + b * CH
            pltpu.sync_copy(src_h.at[pl.ds(eb, CH)], srcv[b])
            pltpu.sync_copy(dst_h.at[pl.ds(eb, CH)], dstv[b])
            pltpu.sync_copy(ef_h.at[pl.ds(eb, CH), :], aug[b])
            pltpu.async_copy(feats_h.at[srcv[b]], rows[b], sems[b])

        def _group(j, carry):
            for b in range(NBUF):
                eb = ebase + (j * NBUF + b + NBUF) * CH
                pltpu.make_async_copy(feats_h.at[srcv[b]], rows[b],
                                      sems[b]).wait()
                pltpu.sync_copy(rows[b], acc_s.at[dstv[b]], add=True)
                pltpu.sync_copy(aug[b], acc_e.at[dstv[b]], add=True)
                pltpu.sync_copy(src_h.at[pl.ds(eb, CH)], srcv[b])
                pltpu.sync_copy(dst_h.at[pl.ds(eb, CH)], dstv[b])
                pltpu.sync_copy(ef_h.at[pl.ds(eb, CH), :], aug[b])
                pltpu.async_copy(feats_h.at[srcv[b]], rows[b], sems[b])
            return carry
        lax.fori_loop(0, NGRP - 1, _group, 0)

        for b in range(NBUF):
            pltpu.make_async_copy(feats_h.at[srcv[b]], rows[b],
                                  sems[b]).wait()
            pltpu.sync_copy(rows[b], acc_s.at[dstv[b]], add=True)
            pltpu.sync_copy(aug[b], acc_e.at[dstv[b]], add=True)
        plsc.subcore_barrier()

        pltpu.sync_copy(acc_s.at[pl.ds(nbase, NPT), :],
                        acc_s_out.at[et, c, pl.ds(nbase, NPT), :])
        pltpu.sync_copy(acc_e.at[pl.ds(nbase, NPT), :],
                        acc_e_out.at[et, c, pl.ds(nbase, NPT), :])


_sc_accumulate = functools.partial(
    pl.kernel,
    out_type=(jax.ShapeDtypeStruct((5, 2, N_PAD, D), jnp.float32),
              jax.ShapeDtypeStruct((5, 2, N_PAD, 16), jnp.float32)),
    mesh=plsc.VectorSubcoreMesh(core_axis_name="c", subcore_axis_name="s"),
    compiler_params=pltpu.CompilerParams(use_tc_tiling_on_sc=False),
    scratch_types=(
        [pltpu.VMEM_SHARED((N_PAD, D), jnp.float32),
         pltpu.VMEM_SHARED((N_PAD, 16), jnp.float32)]
        + [pltpu.VMEM((CH,), jnp.int32) for _ in range(NBUF)]
        + [pltpu.VMEM((CH,), jnp.int32) for _ in range(NBUF)]
        + [pltpu.VMEM((CH, D), jnp.float32) for _ in range(NBUF)]
        + [pltpu.VMEM((CH, 16), jnp.float32) for _ in range(NBUF)]
        + [pltpu.SemaphoreType.DMA for _ in range(NBUF)]
    ),
)(_sc_body)


BN = 1000


def _tc_body(opf, devf, acc_s, acc_e, w_s, w_e, bias, out_op, out_dev):
    outs = []
    for et in range(5):
        a_s = acc_s[et, 0] + acc_s[et, 1]
        a_e = acc_e[et, 0] + acc_e[et, 1]
        cnt = a_e[:, 4:5]
        inv = 1.0 / jnp.maximum(cnt, 1.0)
        o = (jnp.dot(a_s * inv, w_s[et], preferred_element_type=jnp.float32,
                     precision=lax.Precision.HIGHEST)
             + jnp.dot(a_e * inv, w_e[et], preferred_element_type=jnp.float32,
                       precision=lax.Precision.HIGHEST)
             + jnp.where(cnt > 0, jnp.float32(1.0), jnp.float32(0.0)) * bias[et])
        outs.append(o)
    op_agg = (outs[1] + outs[2] + outs[4]) * jnp.float32(1.0 / 3.0)
    dev_agg = (outs[0] + outs[3]) * jnp.float32(0.5)
    out_op[:, :D] = opf[...]
    out_op[:, D:] = jnp.maximum(op_agg, 0.0)
    out_dev[:, :D] = devf[...]
    out_dev[:, D:] = jnp.maximum(dev_agg, 0.0)


def _tc_finalize(opf, devf, acc_s, acc_e, w_s, w_e, bias):
    grid = (N_NODES // BN,)
    return pl.pallas_call(
        _tc_body,
        grid=grid,
        in_specs=[
            pl.BlockSpec((BN, D), lambda i: (i, 0)),
            pl.BlockSpec((BN, D), lambda i: (i, 0)),
            pl.BlockSpec((5, 2, BN, D), lambda i: (0, 0, i, 0)),
            pl.BlockSpec((5, 2, BN, 16), lambda i: (0, 0, i, 0)),
            pl.BlockSpec((5, D, D), lambda i: (0, 0, 0)),
            pl.BlockSpec((5, 16, D), lambda i: (0, 0, 0)),
            pl.BlockSpec((5, D), lambda i: (0, 0)),
        ],
        out_specs=[
            pl.BlockSpec((BN, 2 * D), lambda i: (i, 0)),
            pl.BlockSpec((BN, 2 * D), lambda i: (i, 0)),
        ],
        out_shape=[
            jax.ShapeDtypeStruct((N_NODES, 2 * D), jnp.float32),
            jax.ShapeDtypeStruct((N_NODES, 2 * D), jnp.float32),
        ],
    )(opf, devf, acc_s, acc_e, w_s, w_e, bias)


def kernel(op_feats, device_feats,
           src_link, dst_link, edge_feats_link, W_link, b_link,
           src_prev, dst_prev, edge_feats_prev, W_prev, b_prev,
           src_succ, dst_succ, edge_feats_succ, W_succ, b_succ,
           src_place, dst_place, edge_feats_place, W_place, b_place,
           src_serve, dst_serve, edge_feats_serve, W_serve, b_serve):
    def _aug(ef):
        one = jnp.ones((E_TOTAL, 1), jnp.float32)
        zpad = jnp.zeros((E_TOTAL, 11), jnp.float32)
        return jnp.concatenate([ef, one, zpad], axis=1)

    acc_s, acc_e = _sc_accumulate(
        op_feats, device_feats,
        src_link, dst_link, _aug(edge_feats_link),
        src_prev, dst_prev, _aug(edge_feats_prev),
        src_succ, dst_succ, _aug(edge_feats_succ),
        src_place, dst_place, _aug(edge_feats_place),
        src_serve, dst_serve, _aug(edge_feats_serve))

    ws = (W_link, W_prev, W_succ, W_place, W_serve)
    w_s = jnp.stack([w[:D] for w in ws])
    w_e = jnp.stack([jnp.zeros((16, D), jnp.float32).at[:4].set(w[D:])
                     for w in ws])
    bias = jnp.stack((b_link, b_prev, b_succ, b_place, b_serve))

    out_op, out_dev = _tc_finalize(op_feats, device_feats, acc_s, acc_e,
                                   w_s, w_e, bias)
    return (out_op, out_dev)


# speedup vs baseline: 1.3677x; 1.3677x over previous
import functools

import jax
import jax.numpy as jnp
from jax import lax
from jax.experimental import pallas as pl
from jax.experimental.pallas import tpu as pltpu
from jax.experimental.pallas import tpu_sc as plsc

N_NODES = 10000
N_PAD = 10240
E_TOTAL = 320000
D = 128
NW = 32
EPW = E_TOTAL // NW
CH = 80
NCH = EPW // CH
NPT = N_PAD // 16

_SRC_TAB = (1, 0, 0, 0, 1)


NBUF = 3
NGRP = NCH // NBUF
REM = NCH - NGRP * NBUF


def _sc_body(op_feats, device_feats,
             src0, dst0, ef0, src1, dst1, ef1, src2, dst2, ef2,
             src3, dst3, ef3, src4, dst4, ef4,
             acc_s_out, acc_e_out, *scr):
    acc_s, acc_e = scr[0], scr[1]
    srcv = scr[2:2 + NBUF]
    dstv = scr[2 + NBUF:2 + 2 * NBUF]
    rows = scr[2 + 2 * NBUF:2 + 3 * NBUF]
    aug = scr[2 + 3 * NBUF:2 + 4 * NBUF]
    sems = scr[2 + 4 * NBUF:2 + 5 * NBUF]

    c = lax.axis_index("c")
    s = lax.axis_index("s")
    wid = c * 16 + s
    ebase = wid * EPW
    nbase = s * NPT

    zero16 = jnp.zeros((16,), jnp.float32)

    edge_lists = ((src0, dst0, ef0), (src1, dst1, ef1), (src2, dst2, ef2),
                  (src3, dst3, ef3), (src4, dst4, ef4))

    for et in range(5):
        src_h, dst_h, ef_h = edge_lists[et]
        feats_h = device_feats if _SRC_TAB[et] else op_feats

        def _zrow(i, carry):
            for j in range(8):
                rows[0][i, pl.ds(j * 16, 16)] = zero16
            aug[0][i, :] = zero16
            return carry
        lax.fori_loop(0, CH, _zrow, 0)
        for z in range(NPT // CH):
            pltpu.sync_copy(rows[0], acc_s.at[pl.ds(nbase + z * CH, CH), :])
            pltpu.sync_copy(aug[0], acc_e.at[pl.ds(nbase + z * CH, CH), :])
        plsc.subcore_barrier()

        for b in range(NBUF):
            eb = ebase + b * CH
            pltpu.sync_copy(src_h.at[pl.ds(eb, CH)], srcv[b])
            pltpu.sync_copy(dst_h.at[pl.ds(eb, CH)], dstv[b])
            pltpu.sync_copy(ef_h.at[pl.ds(eb, CH), :], aug[b])
            pltpu.async_copy(feats_h.at[srcv[b]], rows[b], sems[b])

        def _group(j, carry):
            for b in range(NBUF):
                nxt = j * NBUF + b + NBUF
                eb = ebase + nxt * CH
                pltpu.make_async_copy(feats_h.at[srcv[b]], rows[b],
                                      sems[b]).wait()
                pltpu.sync_copy(rows[b], acc_s.at[dstv[b]], add=True)
                pltpu.sync_copy(aug[b], acc_e.at[dstv[b]], add=True)

                @pl.when(nxt < NCH)
                def _():
                    pltpu.sync_copy(src_h.at[pl.ds(eb, CH)], srcv[b])
                    pltpu.sync_copy(dst_h.at[pl.ds(eb, CH)], dstv[b])
                    pltpu.sync_copy(ef_h.at[pl.ds(eb, CH), :], aug[b])
                    pltpu.async_copy(feats_h.at[srcv[b]], rows[b], sems[b])
            return carry
        lax.fori_loop(0, NGRP, _group, 0)

        for b in range(REM):
            pltpu.make_async_copy(feats_h.at[srcv[b]], rows[b],
                                  sems[b]).wait()
            pltpu.sync_copy(rows[b], acc_s.at[dstv[b]], add=True)
            pltpu.sync_copy(aug[b], acc_e.at[dstv[b]], add=True)
        plsc.subcore_barrier()

        pltpu.sync_copy(acc_s.at[pl.ds(nbase, NPT), :],
                        acc_s_out.at[et, c, pl.ds(nbase, NPT), :])
        pltpu.sync_copy(acc_e.at[pl.ds(nbase, NPT), :],
                        acc_e_out.at[et, c, pl.ds(nbase, NPT), :])


_sc_accumulate = functools.partial(
    pl.kernel,
    out_type=(jax.ShapeDtypeStruct((5, 2, N_PAD, D), jnp.float32),
              jax.ShapeDtypeStruct((5, 2, N_PAD, 16), jnp.float32)),
    mesh=plsc.VectorSubcoreMesh(core_axis_name="c", subcore_axis_name="s"),
    compiler_params=pltpu.CompilerParams(use_tc_tiling_on_sc=False),
    scratch_types=(
        [pltpu.VMEM_SHARED((N_PAD, D), jnp.float32),
         pltpu.VMEM_SHARED((N_PAD, 16), jnp.float32)]
        + [pltpu.VMEM((CH,), jnp.int32) for _ in range(NBUF)]
        + [pltpu.VMEM((CH,), jnp.int32) for _ in range(NBUF)]
        + [pltpu.VMEM((CH, D), jnp.float32) for _ in range(NBUF)]
        + [pltpu.VMEM((CH, 16), jnp.float32) for _ in range(NBUF)]
        + [pltpu.SemaphoreType.DMA for _ in range(NBUF)]
    ),
)(_sc_body)


BN = 1000


def _tc_body(opf, devf, acc_s, acc_e, w_s, w_e, bias, out_op, out_dev):
    outs = []
    for et in range(5):
        a_s = acc_s[et, 0] + acc_s[et, 1]
        a_e = acc_e[et, 0] + acc_e[et, 1]
        cnt = a_e[:, 4:5]
        inv = 1.0 / jnp.maximum(cnt, 1.0)
        o = (jnp.dot(a_s * inv, w_s[et], preferred_element_type=jnp.float32,
                     precision=lax.Precision.HIGHEST)
             + jnp.dot(a_e * inv, w_e[et], preferred_element_type=jnp.float32,
                       precision=lax.Precision.HIGHEST)
             + jnp.where(cnt > 0, jnp.float32(1.0), jnp.float32(0.0)) * bias[et])
        outs.append(o)
    op_agg = (outs[1] + outs[2] + outs[4]) * jnp.float32(1.0 / 3.0)
    dev_agg = (outs[0] + outs[3]) * jnp.float32(0.5)
    out_op[:, :D] = opf[...]
    out_op[:, D:] = jnp.maximum(op_agg, 0.0)
    out_dev[:, :D] = devf[...]
    out_dev[:, D:] = jnp.maximum(dev_agg, 0.0)


def _tc_finalize(opf, devf, acc_s, acc_e, w_s, w_e, bias):
    grid = (N_NODES // BN,)
    return pl.pallas_call(
        _tc_body,
        grid=grid,
        in_specs=[
            pl.BlockSpec((BN, D), lambda i: (i, 0)),
            pl.BlockSpec((BN, D), lambda i: (i, 0)),
            pl.BlockSpec((5, 2, BN, D), lambda i: (0, 0, i, 0)),
            pl.BlockSpec((5, 2, BN, 16), lambda i: (0, 0, i, 0)),
            pl.BlockSpec((5, D, D), lambda i: (0, 0, 0)),
            pl.BlockSpec((5, 16, D), lambda i: (0, 0, 0)),
            pl.BlockSpec((5, D), lambda i: (0, 0)),
        ],
        out_specs=[
            pl.BlockSpec((BN, 2 * D), lambda i: (i, 0)),
            pl.BlockSpec((BN, 2 * D), lambda i: (i, 0)),
        ],
        out_shape=[
            jax.ShapeDtypeStruct((N_NODES, 2 * D), jnp.float32),
            jax.ShapeDtypeStruct((N_NODES, 2 * D), jnp.float32),
        ],
    )(opf, devf, acc_s, acc_e, w_s, w_e, bias)


def kernel(op_feats, device_feats,
           src_link, dst_link, edge_feats_link, W_link, b_link,
           src_prev, dst_prev, edge_feats_prev, W_prev, b_prev,
           src_succ, dst_succ, edge_feats_succ, W_succ, b_succ,
           src_place, dst_place, edge_feats_place, W_place, b_place,
           src_serve, dst_serve, edge_feats_serve, W_serve, b_serve):
    def _aug(ef):
        one = jnp.ones((E_TOTAL, 1), jnp.float32)
        zpad = jnp.zeros((E_TOTAL, 11), jnp.float32)
        return jnp.concatenate([ef, one, zpad], axis=1)

    acc_s, acc_e = _sc_accumulate(
        op_feats, device_feats,
        src_link, dst_link, _aug(edge_feats_link),
        src_prev, dst_prev, _aug(edge_feats_prev),
        src_succ, dst_succ, _aug(edge_feats_succ),
        src_place, dst_place, _aug(edge_feats_place),
        src_serve, dst_serve, _aug(edge_feats_serve))

    ws = (W_link, W_prev, W_succ, W_place, W_serve)
    w_s = jnp.stack([w[:D] for w in ws])
    w_e = jnp.stack([jnp.zeros((16, D), jnp.float32).at[:4].set(w[D:])
                     for w in ws])
    bias = jnp.stack((b_link, b_prev, b_succ, b_place, b_serve))

    out_op, out_dev = _tc_finalize(op_feats, device_feats, acc_s, acc_e,
                                   w_s, w_e, bias)
    return (out_op, out_dev)
